# Initial kernel scaffold; baseline (speedup 1.0000x reference)
import functools

import jax
import jax.numpy as jnp
from jax import lax
from jax.experimental import pallas as pl
from jax.experimental.pallas import tpu as pltpu
from jax.experimental.pallas import tpu_sc as plsc

_HOPS = 4
_ALPHA = 0.5
_K = 80


@functools.cache
def _make_scatter(N, D, C, K, NC, NS):
    NW = NC * NS
    RPT = N // NS
    ZR = 125 if RPT % 125 == 0 else RPT

    mesh = plsc.VectorSubcoreMesh(core_axis_name="c", subcore_axis_name="s")

    def body(rows_hbm, cols_hbm, vals_hbm, h_hbm, out_hbm,
             rows_v, cols_v, vals_v, gath_v, zbuf_v, agg_sh):
        c = lax.axis_index("c")
        s = lax.axis_index("s")
        wid = s * NC + c

        zeros16 = jnp.zeros((16,), jnp.float32)

        def zrow(r, carry):
            for j in range(D // 16):
                zbuf_v[r, pl.ds(j * 16, 16)] = zeros16
            return carry

        lax.fori_loop(0, ZR, zrow, 0)
        for i in range(RPT // ZR):
            pltpu.sync_copy(zbuf_v, agg_sh.at[pl.ds(s * RPT + i * ZR, ZR)])
        plsc.subcore_barrier()

        pltpu.sync_copy(rows_hbm.at[wid], rows_v)
        pltpu.sync_copy(cols_hbm.at[wid], cols_v)
        pltpu.sync_copy(vals_hbm.at[wid], vals_v)

        def chunk(t, carry):
            pltpu.sync_copy(h_hbm.at[cols_v.at[t]], gath_v)

            def scale(e, carry2):
                v = vals_v[t, e]
                for j in range(D // 16):
                    sl = pl.ds(j * 16, 16)
                    gath_v[e, sl] = gath_v[e, sl] * v
                return carry2

            lax.fori_loop(0, K, scale, 0)

            pltpu.sync_copy(gath_v, agg_sh.at[rows_v.at[t]], add=True)
            return carry

        lax.fori_loop(0, C, chunk, 0)

        plsc.subcore_barrier()
        pltpu.sync_copy(agg_sh.at[pl.ds(s * RPT, RPT)],
                        out_hbm.at[c, pl.ds(s * RPT, RPT)])

    return pl.kernel(
        body,
        out_type=jax.ShapeDtypeStruct((NC, N, D), jnp.float32),
        mesh=mesh,
        scratch_types=[
            pltpu.VMEM((C, K), jnp.int32),
            pltpu.VMEM((C, K), jnp.int32),
            pltpu.VMEM((C, K), jnp.float32),
            pltpu.VMEM((K, D), jnp.float32),
            pltpu.VMEM((ZR, D), jnp.float32),
            pltpu.VMEM_SHARED((N, D), jnp.float32),
        ],
    )


@functools.cache
def _make_combine(N, D, NC):
    BR = 1000 if N % 1000 == 0 else N

    def body(h_ref, p_ref, o_ref):
        agg = p_ref[0]
        for c in range(1, NC):
            agg = agg + p_ref[c]
        o_ref[...] = _ALPHA * h_ref[...] + (1.0 - _ALPHA) * agg

    return pl.pallas_call(
        body,
        grid=(N // BR,),
        in_specs=[
            pl.BlockSpec((BR, D), lambda i: (i, 0)),
            pl.BlockSpec((NC, BR, D), lambda i: (0, i, 0)),
        ],
        out_specs=pl.BlockSpec((BR, D), lambda i: (i, 0)),
        out_shape=jax.ShapeDtypeStruct((N, D), jnp.float32),
    )


def kernel(H, rows, cols, vals):
    N, D = H.shape
    E = rows.shape[0]
    info = plsc.get_sparse_core_info()
    NC, NS = info.num_cores, info.num_subcores
    NW = NC * NS
    CH = NW * _K
    Epad = ((E + CH - 1) // CH) * CH
    if Epad != E:
        pad = Epad - E
        rows = jnp.concatenate([rows, jnp.zeros((pad,), rows.dtype)])
        cols = jnp.concatenate([cols, jnp.zeros((pad,), cols.dtype)])
        vals = jnp.concatenate([vals, jnp.zeros((pad,), vals.dtype)])
    C = Epad // CH
    rows3 = rows.reshape(NW, C, _K)
    cols3 = cols.reshape(NW, C, _K)
    vals3 = vals.reshape(NW, C, _K)

    scatter = _make_scatter(N, D, C, _K, NC, NS)
    combine = _make_combine(N, D, NC)

    h = H.astype(jnp.float32)
    for _ in range(_HOPS):
        partials = scatter(rows3, cols3, vals3, h)
        h = combine(h, partials)
    return h.astype(H.dtype)


# speedup vs baseline: 2.7939x; 2.7939x over previous
import functools

import jax
import jax.numpy as jnp
from jax import lax
from jax.experimental import pallas as pl
from jax.experimental.pallas import tpu as pltpu
from jax.experimental.pallas import tpu_sc as plsc

_HOPS = 4
_ALPHA = 0.5
_K = 80


@functools.cache
def _make_diffusion(Npad, HD, C, K, NC, NS):
    RPT = Npad // NS
    ZR = 128
    ZFULL, ZTAIL = divmod(RPT, ZR)
    NJ = HD // 16

    mesh = plsc.VectorSubcoreMesh(core_axis_name="c", subcore_axis_name="s")

    def body(rows_hbm, cols_hbm, vals_hbm, h0_hbm, outA_hbm, outB_hbm,
             rows_v, cols_v, vals_v, gath_v, zbuf_v, hbuf_v, abuf_v, agg_sh):
        c = lax.axis_index("c")
        s = lax.axis_index("s")

        zeros16 = jnp.zeros((16,), jnp.float32)

        def zrow(r, carry):
            for j in range(NJ):
                zbuf_v[r, pl.ds(j * 16, 16)] = zeros16
            return carry

        lax.fori_loop(0, ZR, zrow, 0)

        def zero_agg():
            for i in range(ZFULL):
                pltpu.sync_copy(zbuf_v, agg_sh.at[pl.ds(s * RPT + i * ZR, ZR)])
            if ZTAIL:
                pltpu.sync_copy(zbuf_v.at[pl.ds(0, ZTAIL)],
                                agg_sh.at[pl.ds(s * RPT + ZFULL * ZR, ZTAIL)])

        zero_agg()

        pltpu.sync_copy(rows_hbm.at[s], rows_v)
        pltpu.sync_copy(cols_hbm.at[s], cols_v)
        pltpu.sync_copy(vals_hbm.at[s], vals_v)
        plsc.subcore_barrier()

        def hop(src_hbm, dst_hbm):
            src_c = src_hbm.at[c]
            dst_c = dst_hbm.at[c]

            def chunk(t, carry):
                pltpu.sync_copy(src_c.at[cols_v.at[t]], gath_v)

                def scale(g, carry2):
                    vvec = vals_v[t, pl.ds(g * 16, 16)]
                    for lane in range(16):
                        e = g * 16 + lane
                        v = vvec[lane]
                        for j in range(NJ):
                            sl = pl.ds(j * 16, 16)
                            gath_v[e, sl] = gath_v[e, sl] * v
                    return carry2

                lax.fori_loop(0, K // 16, scale, 0)

                pltpu.sync_copy(gath_v, agg_sh.at[rows_v.at[t]], add=True)
                return carry

            lax.fori_loop(0, C, chunk, 0)
            plsc.subcore_barrier()

            def blend_block(row0, nrows):
                pltpu.sync_copy(src_c.at[pl.ds(row0, nrows)],
                                hbuf_v.at[pl.ds(0, nrows)])
                pltpu.sync_copy(agg_sh.at[pl.ds(row0, nrows)],
                                abuf_v.at[pl.ds(0, nrows)])

                def brow(r, carry):
                    for j in range(NJ):
                        sl = pl.ds(j * 16, 16)
                        hbuf_v[r, sl] = (_ALPHA * hbuf_v[r, sl]
                                         + (1.0 - _ALPHA) * abuf_v[r, sl])
                    return carry

                lax.fori_loop(0, nrows, brow, 0)
                pltpu.sync_copy(hbuf_v.at[pl.ds(0, nrows)],
                                dst_c.at[pl.ds(row0, nrows)])
                pltpu.sync_copy(zbuf_v.at[pl.ds(0, nrows)],
                                agg_sh.at[pl.ds(row0, nrows)])

            for i in range(ZFULL):
                blend_block(s * RPT + i * ZR, ZR)
            if ZTAIL:
                blend_block(s * RPT + ZFULL * ZR, ZTAIL)
            plsc.subcore_barrier()

        src = h0_hbm
        bufs = (outA_hbm, outB_hbm)
        for h in range(_HOPS):
            dst = bufs[h % 2]
            hop(src, dst)
            src = dst

    return pl.kernel(
        body,
        out_type=(jax.ShapeDtypeStruct((NC, Npad, HD), jnp.float32),
                  jax.ShapeDtypeStruct((NC, Npad, HD), jnp.float32)),
        mesh=mesh,
        compiler_params=pltpu.CompilerParams(use_tc_tiling_on_sc=False),
        scratch_types=[
            pltpu.VMEM((C, K), jnp.int32),
            pltpu.VMEM((C, K), jnp.int32),
            pltpu.VMEM((C, K), jnp.float32),
            pltpu.VMEM((K, HD), jnp.float32),
            pltpu.VMEM((ZR, HD), jnp.float32),
            pltpu.VMEM((ZR, HD), jnp.float32),
            pltpu.VMEM((ZR, HD), jnp.float32),
            pltpu.VMEM_SHARED((Npad, HD), jnp.float32),
        ],
    )


def kernel(H, rows, cols, vals):
    N, D = H.shape
    E = rows.shape[0]
    info = plsc.get_sparse_core_info()
    NC, NS = info.num_cores, info.num_subcores
    HD = D // NC

    CH = NS * _K
    Epad = ((E + CH - 1) // CH) * CH
    if Epad != E:
        pad = Epad - E
        rows = jnp.concatenate([rows, jnp.zeros((pad,), rows.dtype)])
        cols = jnp.concatenate([cols, jnp.zeros((pad,), cols.dtype)])
        vals = jnp.concatenate([vals, jnp.zeros((pad,), vals.dtype)])
    C = Epad // CH
    rows3 = rows.reshape(NS, C, _K)
    cols3 = cols.reshape(NS, C, _K)
    vals3 = vals.reshape(NS, C, _K)

    NCH = NS * 8
    Npad = ((N + NCH - 1) // NCH) * NCH

    h32 = H.astype(jnp.float32)
    h0 = jnp.pad(h32, ((0, Npad - N), (0, 0)))
    h0 = h0.reshape(Npad, NC, HD).transpose(1, 0, 2)

    diffuse = _make_diffusion(Npad, HD, C, _K, NC, NS)
    outA, outB = diffuse(rows3, cols3, vals3, h0)
    final = outB if _HOPS % 2 == 0 else outA
    out = final[:, :N, :].transpose(1, 0, 2).reshape(N, D)
    return out.astype(H.dtype)


# speedup vs baseline: 9.9916x; 3.5762x over previous
import functools

import jax
import jax.numpy as jnp
from jax import lax
from jax.experimental import pallas as pl
from jax.experimental.pallas import tpu as pltpu
from jax.experimental.pallas import tpu_sc as plsc

_HOPS = 4
_ALPHA = 0.5
_K = 80
_SHIFT = 14
_MASK = (1 << _SHIFT) - 1


@functools.cache
def _make_diffusion(Npad, HD, C, K, NC, NS):
    RPT = Npad // NS
    ZR = 128
    ZFULL, ZTAIL = divmod(RPT, ZR)
    NJ = HD // 16

    mesh = plsc.VectorSubcoreMesh(core_axis_name="c", subcore_axis_name="s")

    def body(packed_hbm, vals_hbm, h0_hbm, outA_hbm, outB_hbm,
             packed_v, vals_v, colb_v, rowb_v, gath_v, scaled_v,
             zbuf_v, hbuf_v, abuf_v, agg_sh, gsem, ssem):
        c = lax.axis_index("c")
        s = lax.axis_index("s")

        zeros16 = jnp.zeros((16,), jnp.float32)

        def zrow(r, carry):
            for j in range(NJ):
                zbuf_v[r, pl.ds(j * 16, 16)] = zeros16
            return carry

        lax.fori_loop(0, ZR, zrow, 0)

        def zero_agg():
            for i in range(ZFULL):
                pltpu.sync_copy(zbuf_v, agg_sh.at[pl.ds(s * RPT + i * ZR, ZR)])
            if ZTAIL:
                pltpu.sync_copy(zbuf_v.at[pl.ds(0, ZTAIL)],
                                agg_sh.at[pl.ds(s * RPT + ZFULL * ZR, ZTAIL)])

        zero_agg()

        pltpu.sync_copy(packed_hbm.at[s], packed_v)
        pltpu.sync_copy(vals_hbm.at[s], vals_v)
        plsc.subcore_barrier()

        NB = 2

        def hop(src_hbm, dst_hbm):
            src_c = src_hbm.at[c]
            dst_c = dst_hbm.at[c]

            def unpack_cols(t, b):
                for g in range(K // 16):
                    sl = pl.ds(g * 16, 16)
                    colb_v[b, sl] = packed_v[t, sl] & _MASK

            def unpack_rows(t, b):
                for g in range(K // 16):
                    sl = pl.ds(g * 16, 16)
                    rowb_v[b, sl] = lax.shift_right_logical(
                        packed_v[t, sl], _SHIFT)

            def start_gather(b):
                pltpu.async_copy(src_c.at[colb_v.at[b]], gath_v.at[b],
                                 gsem[b])

            def wait_gather(b):
                pltpu.make_async_copy(src_c.at[pl.ds(0, K)], gath_v.at[b],
                                      gsem[b]).wait()

            def start_scatter(b):
                pltpu.async_copy(scaled_v.at[b], agg_sh.at[rowb_v.at[b]],
                                 ssem[b], add=True)

            def wait_scatter(b):
                pltpu.make_async_copy(scaled_v.at[b],
                                      agg_sh.at[rowb_v.at[b]],
                                      ssem[b]).wait()

            for b in range(NB):
                unpack_cols(b, b)
                start_gather(b)

            def outer(tt, carry):
                for b in range(NB):
                    t = tt * NB + b
                    wait_gather(b)

                    @pl.when(t >= NB)
                    def _():
                        wait_scatter(b)

                    unpack_rows(t, b)

                    def scale(g, carry2):
                        vvec = vals_v[t, pl.ds(g * 16, 16)]
                        for lane in range(16):
                            e = g * 16 + lane
                            v = vvec[lane]
                            for j in range(NJ):
                                sl = pl.ds(j * 16, 16)
                                scaled_v[b, e, sl] = gath_v[b, e, sl] * v
                        return carry2

                    lax.fori_loop(0, K // 16, scale, 0)

                    start_scatter(b)

                    @pl.when(t + NB < C)
                    def _():
                        unpack_cols(t + NB, b)
                        start_gather(b)

                return carry

            lax.fori_loop(0, C // NB, outer, 0)

            for b in range(NB):
                wait_scatter(b)
            plsc.subcore_barrier()

            def blend_block(row0, nrows):
                pltpu.sync_copy(src_c.at[pl.ds(row0, nrows)],
                                hbuf_v.at[pl.ds(0, nrows)])
                pltpu.sync_copy(agg_sh.at[pl.ds(row0, nrows)],
                                abuf_v.at[pl.ds(0, nrows)])

                def brow(r, carry):
                    for j in range(NJ):
                        sl = pl.ds(j * 16, 16)
                        hbuf_v[r, sl] = (_ALPHA * hbuf_v[r, sl]
                                         + (1.0 - _ALPHA) * abuf_v[r, sl])
                    return carry

                lax.fori_loop(0, nrows, brow, 0)
                pltpu.sync_copy(hbuf_v.at[pl.ds(0, nrows)],
                                dst_c.at[pl.ds(row0, nrows)])
                pltpu.sync_copy(zbuf_v.at[pl.ds(0, nrows)],
                                agg_sh.at[pl.ds(row0, nrows)])

            for i in range(ZFULL):
                blend_block(s * RPT + i * ZR, ZR)
            if ZTAIL:
                blend_block(s * RPT + ZFULL * ZR, ZTAIL)
            plsc.subcore_barrier()

        src = h0_hbm
        bufs = (outA_hbm, outB_hbm)
        for h in range(_HOPS):
            dst = bufs[h % 2]
            hop(src, dst)
            src = dst

    return pl.kernel(
        body,
        out_type=(jax.ShapeDtypeStruct((NC, Npad, HD), jnp.float32),
                  jax.ShapeDtypeStruct((NC, Npad, HD), jnp.float32)),
        mesh=mesh,
        compiler_params=pltpu.CompilerParams(use_tc_tiling_on_sc=False),
        scratch_types=[
            pltpu.VMEM((C, K), jnp.int32),
            pltpu.VMEM((C, K), jnp.float32),
            pltpu.VMEM((2, K), jnp.int32),
            pltpu.VMEM((2, K), jnp.int32),
            pltpu.VMEM((2, K, HD), jnp.float32),
            pltpu.VMEM((2, K, HD), jnp.float32),
            pltpu.VMEM((ZR, HD), jnp.float32),
            pltpu.VMEM((ZR, HD), jnp.float32),
            pltpu.VMEM((ZR, HD), jnp.float32),
            pltpu.VMEM_SHARED((Npad, HD), jnp.float32),
            (pltpu.SemaphoreType.DMA, pltpu.SemaphoreType.DMA),
            (pltpu.SemaphoreType.DMA, pltpu.SemaphoreType.DMA),
        ],
    )


def kernel(H, rows, cols, vals):
    N, D = H.shape
    E = rows.shape[0]
    info = plsc.get_sparse_core_info()
    NC, NS = info.num_cores, info.num_subcores
    HD = D // NC

    CH = NS * _K
    Epad = ((E + CH - 1) // CH) * CH
    if Epad != E:
        pad = Epad - E
        rows = jnp.concatenate([rows, jnp.zeros((pad,), rows.dtype)])
        cols = jnp.concatenate([cols, jnp.zeros((pad,), cols.dtype)])
        vals = jnp.concatenate([vals, jnp.zeros((pad,), vals.dtype)])
    C = Epad // CH
    assert N <= (1 << _SHIFT)
    packed = (rows.astype(jnp.int32) << _SHIFT) | cols.astype(jnp.int32)
    packed3 = packed.reshape(NS, C, _K)
    vals3 = vals.reshape(NS, C, _K)

    NCH = NS * 8
    Npad = ((N + NCH - 1) // NCH) * NCH

    h32 = H.astype(jnp.float32)
    h0 = jnp.pad(h32, ((0, Npad - N), (0, 0)))
    h0 = h0.reshape(Npad, NC, HD).transpose(1, 0, 2)

    diffuse = _make_diffusion(Npad, HD, C, _K, NC, NS)
    outA, outB = diffuse(packed3, vals3, h0)
    final = outB if _HOPS % 2 == 0 else outA
    out = final[:, :N, :].transpose(1, 0, 2).reshape(N, D)
    return out.astype(H.dtype)
